# baseline (speedup 1.0000x reference)
import jax
import jax.numpy as jnp
import numpy as np
from jax.experimental import pallas as pl
from jax.experimental.pallas import tpu as pltpu

_ROWS, _COLS = 4096, 768
_SIZE = 3

_key = jax.random.key(42)
_k1, _k2 = jax.random.split(_key)
_R_IDX = int(jax.random.randint(_k1, (), 0, _ROWS - _SIZE))
_PERM = [int(v) for v in np.asarray(jax.random.permutation(_k2, _SIZE))]

_P0 = 8 * (_R_IDX // 8)
_P1 = 8 * (-(-(_R_IDX + _SIZE) // 8))
_PSZ = _P1 - _P0
_PATCH_SRC = list(range(_PSZ))
for _j in range(_SIZE):
    _PATCH_SRC[_R_IDX - _P0 + _j] = _R_IDX - _P0 + _PERM[_j]

_TARGET_CHUNK = 512


def _segments():
    segs = []
    for lo, hi in ((0, _P0), (_P1, _ROWS)):
        n = hi - lo
        if n <= 0:
            continue
        k = max(1, round(n / _TARGET_CHUNK))
        step = 8 * (-(-n // (8 * k)))
        for s in range(lo, hi, step):
            segs.append((s, min(step, hi - s)))
    return segs


_SEGS = _segments()


def _body(x_ref, o_ref, vin, vout, seg_sem, in_sem, out_sem):
    segs = [
        pltpu.make_async_copy(
            x_ref.at[pl.ds(start, n)], o_ref.at[pl.ds(start, n)], seg_sem)
        for start, n in _SEGS
    ]
    for c in segs:
        c.start()

    patch_in = pltpu.make_async_copy(
        x_ref.at[pl.ds(_P0, _PSZ)], vin, in_sem)
    patch_in.start()
    patch_in.wait()
    for j in range(_PSZ):
        s = _PATCH_SRC[j]
        vout[j:j + 1, :] = vin[s:s + 1, :]
    patch_out = pltpu.make_async_copy(
        vout, o_ref.at[pl.ds(_P0, _PSZ)], out_sem)
    patch_out.start()
    patch_out.wait()

    for c in segs:
        c.wait()


def kernel(x, y):
    x_out = pl.pallas_call(
        _body,
        in_specs=[pl.BlockSpec(memory_space=pltpu.MemorySpace.HBM)],
        out_specs=pl.BlockSpec(memory_space=pltpu.MemorySpace.HBM),
        out_shape=jax.ShapeDtypeStruct((_ROWS, _COLS), jnp.float32),
        scratch_shapes=[
            pltpu.VMEM((_PSZ, _COLS), jnp.float32),
            pltpu.VMEM((_PSZ, _COLS), jnp.float32),
            pltpu.SemaphoreType.DMA,
            pltpu.SemaphoreType.DMA,
            pltpu.SemaphoreType.DMA,
        ],
    )(x)
    return (x_out, y)


# speedup vs baseline: 11.8783x; 11.8783x over previous
import functools

import jax
import jax.numpy as jnp
import numpy as np
from jax import lax
from jax.experimental import pallas as pl
from jax.experimental.pallas import tpu as pltpu
from jax.experimental.pallas import tpu_sc as plsc

_ROWS, _COLS = 4096, 768
_SIZE = 3

_key = jax.random.key(42)
_k1, _k2 = jax.random.split(_key)
_R_IDX = int(jax.random.randint(_k1, (), 0, _ROWS - _SIZE))
_PERM = [int(v) for v in np.asarray(jax.random.permutation(_k2, _SIZE))]

_NC, _NS = 2, 16
_NW = _NC * _NS
_RPW = _ROWS // _NW

_OWNERS: dict[int, list[int]] = {}
for _j in range(_SIZE):
    _OWNERS.setdefault((_R_IDX + _j) // _RPW, []).append(_j)

_mesh = plsc.VectorSubcoreMesh(
    core_axis_name="c", subcore_axis_name="s",
    num_cores=_NC, num_subcores=_NS)


@functools.partial(
    pl.kernel,
    out_type=jax.ShapeDtypeStruct((_ROWS, _COLS), jnp.float32),
    mesh=_mesh,
    scratch_types=[pltpu.VMEM((_RPW, _COLS), jnp.float32)],
)
def _sc_permute(x_hbm, o_hbm, slab):
    wid = lax.axis_index("s") * _NC + lax.axis_index("c")
    base = wid * _RPW
    pltpu.sync_copy(x_hbm.at[pl.ds(base, _RPW)], slab)
    for _owner, _js in _OWNERS.items():
        @pl.when(wid == _owner)
        def _(_owner=_owner, _js=_js):
            for j in _js:
                pltpu.sync_copy(
                    x_hbm.at[pl.ds(_R_IDX + _PERM[j], 1)],
                    slab.at[pl.ds(_R_IDX - _owner * _RPW + j, 1)])
    pltpu.sync_copy(slab, o_hbm.at[pl.ds(base, _RPW)])


def kernel(x, y):
    return (_sc_permute(x), y)


# speedup vs baseline: 12.1504x; 1.0229x over previous
import functools

import jax
import jax.numpy as jnp
import numpy as np
from jax import lax
from jax.experimental import pallas as pl
from jax.experimental.pallas import tpu as pltpu
from jax.experimental.pallas import tpu_sc as plsc

_ROWS, _COLS = 4096, 768
_SIZE = 3

_key = jax.random.key(42)
_k1, _k2 = jax.random.split(_key)
_R_IDX = int(jax.random.randint(_k1, (), 0, _ROWS - _SIZE))
_PERM = [int(v) for v in np.asarray(jax.random.permutation(_k2, _SIZE))]

_NC, _NS = 2, 16
_NW = _NC * _NS
_RPW = _ROWS // _NW

_OWNERS: dict[int, list[int]] = {}
for _j in range(_SIZE):
    _OWNERS.setdefault((_R_IDX + _j) // _RPW, []).append(_j)

_mesh = plsc.VectorSubcoreMesh(
    core_axis_name="c", subcore_axis_name="s",
    num_cores=_NC, num_subcores=_NS)


_NCHUNK = 4
_CH = _RPW // _NCHUNK


@functools.partial(
    pl.kernel,
    out_type=jax.ShapeDtypeStruct((_ROWS, _COLS), jnp.float32),
    mesh=_mesh,
    scratch_types=[pltpu.VMEM((_RPW, _COLS), jnp.float32)]
    + [pltpu.SemaphoreType.DMA] * (2 * _NCHUNK),
)
def _sc_permute(x_hbm, o_hbm, slab, *sems):
    in_sems, out_sems = sems[:_NCHUNK], sems[_NCHUNK:]
    wid = lax.axis_index("s") * _NC + lax.axis_index("c")
    base = wid * _RPW
    loads = [
        pltpu.make_async_copy(
            x_hbm.at[pl.ds(base + k * _CH, _CH)],
            slab.at[pl.ds(k * _CH, _CH)], in_sems[k])
        for k in range(_NCHUNK)
    ]
    stores = [
        pltpu.make_async_copy(
            slab.at[pl.ds(k * _CH, _CH)],
            o_hbm.at[pl.ds(base + k * _CH, _CH)], out_sems[k])
        for k in range(_NCHUNK)
    ]
    for c in loads:
        c.start()
    for k in range(_NCHUNK):
        loads[k].wait()
        for _owner, _js in _OWNERS.items():
            _rel = [j for j in _js
                    if (_R_IDX + j) // _CH - _owner * _NCHUNK == k]
            if not _rel:
                continue

            @pl.when(wid == _owner)
            def _(_owner=_owner, _rel=_rel):
                for j in _rel:
                    pltpu.sync_copy(
                        x_hbm.at[pl.ds(_R_IDX + _PERM[j], 1)],
                        slab.at[pl.ds(_R_IDX - _owner * _RPW + j, 1)])
        stores[k].start()
    for c in stores:
        c.wait()


def kernel(x, y):
    return (_sc_permute(x), y)


# speedup vs baseline: 12.1826x; 1.0026x over previous
import functools

import jax
import jax.numpy as jnp
import numpy as np
from jax import lax
from jax.experimental import pallas as pl
from jax.experimental.pallas import tpu as pltpu
from jax.experimental.pallas import tpu_sc as plsc

_ROWS, _COLS = 4096, 768
_SIZE = 3

_key = jax.random.key(42)
_k1, _k2 = jax.random.split(_key)
_R_IDX = int(jax.random.randint(_k1, (), 0, _ROWS - _SIZE))
_PERM = [int(v) for v in np.asarray(jax.random.permutation(_k2, _SIZE))]

_NC, _NS = 2, 16
_NW = _NC * _NS
_RPW = _ROWS // _NW

_OWNERS: dict[int, list[int]] = {}
for _j in range(_SIZE):
    _OWNERS.setdefault((_R_IDX + _j) // _RPW, []).append(_j)

_mesh = plsc.VectorSubcoreMesh(
    core_axis_name="c", subcore_axis_name="s",
    num_cores=_NC, num_subcores=_NS)


_NCHUNK = 4
_CH = _RPW // _NCHUNK


@functools.partial(
    pl.kernel,
    out_type=jax.ShapeDtypeStruct((_ROWS, _COLS), jnp.float32),
    mesh=_mesh,
    scratch_types=[pltpu.VMEM((_RPW, _COLS), jnp.float32)]
    + [pltpu.SemaphoreType.DMA] * (2 * _NCHUNK),
    compiler_params=pltpu.CompilerParams(use_tc_tiling_on_sc=True),
)
def _sc_permute(x_hbm, o_hbm, slab, *sems):
    in_sems, out_sems = sems[:_NCHUNK], sems[_NCHUNK:]
    wid = lax.axis_index("s") * _NC + lax.axis_index("c")
    base = wid * _RPW
    loads = [
        pltpu.make_async_copy(
            x_hbm.at[pl.ds(base + k * _CH, _CH)],
            slab.at[pl.ds(k * _CH, _CH)], in_sems[k])
        for k in range(_NCHUNK)
    ]
    stores = [
        pltpu.make_async_copy(
            slab.at[pl.ds(k * _CH, _CH)],
            o_hbm.at[pl.ds(base + k * _CH, _CH)], out_sems[k])
        for k in range(_NCHUNK)
    ]
    for c in loads:
        c.start()
    for k in range(_NCHUNK):
        loads[k].wait()
        for _owner, _js in _OWNERS.items():
            _rel = [j for j in _js
                    if (_R_IDX + j) // _CH - _owner * _NCHUNK == k]
            if not _rel:
                continue

            @pl.when(wid == _owner)
            def _(_owner=_owner, _rel=_rel):
                for j in _rel:
                    pltpu.sync_copy(
                        x_hbm.at[pl.ds(_R_IDX + _PERM[j], 1)],
                        slab.at[pl.ds(_R_IDX - _owner * _RPW + j, 1)])
        stores[k].start()
    for c in stores:
        c.wait()


def kernel(x, y):
    return (_sc_permute(x), y)


# speedup vs baseline: 12.2161x; 1.0028x over previous
import functools

import jax
import jax.numpy as jnp
import numpy as np
from jax import lax
from jax.experimental import pallas as pl
from jax.experimental.pallas import tpu as pltpu
from jax.experimental.pallas import tpu_sc as plsc

_ROWS, _COLS = 4096, 768
_SIZE = 3

_key = jax.random.key(42)
_k1, _k2 = jax.random.split(_key)
_R_IDX = int(jax.random.randint(_k1, (), 0, _ROWS - _SIZE))
_PERM = [int(v) for v in np.asarray(jax.random.permutation(_k2, _SIZE))]

_NC, _NS = 2, 16
_NW = _NC * _NS
_RPW = _ROWS // _NW

_OWNERS: dict[int, list[int]] = {}
for _j in range(_SIZE):
    _OWNERS.setdefault((_R_IDX + _j) // _RPW, []).append(_j)

_mesh = plsc.VectorSubcoreMesh(
    core_axis_name="c", subcore_axis_name="s",
    num_cores=_NC, num_subcores=_NS)


_NCHUNK = 4
_CH = _RPW // _NCHUNK


@functools.partial(
    pl.kernel,
    out_type=jax.ShapeDtypeStruct((_ROWS, _COLS), jnp.float32),
    mesh=_mesh,
    scratch_types=[pltpu.VMEM((_RPW, _COLS), jnp.float32)]
    + [pltpu.SemaphoreType.DMA] * (2 * _NCHUNK),
)
def _sc_permute(x_hbm, o_hbm, slab, *sems):
    in_sems, out_sems = sems[:_NCHUNK], sems[_NCHUNK:]
    wid = lax.axis_index("s") * _NC + lax.axis_index("c")
    base = wid * _RPW
    loads = [
        pltpu.make_async_copy(
            x_hbm.at[pl.ds(base + k * _CH, _CH)],
            slab.at[pl.ds(k * _CH, _CH)], in_sems[k])
        for k in range(_NCHUNK)
    ]
    stores = [
        pltpu.make_async_copy(
            slab.at[pl.ds(k * _CH, _CH)],
            o_hbm.at[pl.ds(base + k * _CH, _CH)], out_sems[k])
        for k in range(_NCHUNK)
    ]
    for c in loads:
        c.start()
    for k in range(_NCHUNK):
        loads[k].wait()
        for _owner, _js in _OWNERS.items():
            _rel = [j for j in _js
                    if (_R_IDX + j) // _CH - _owner * _NCHUNK == k]
            if not _rel:
                continue

            @pl.when(wid == _owner)
            def _(_owner=_owner, _rel=_rel):
                for j in _rel:
                    pltpu.sync_copy(
                        x_hbm.at[pl.ds(_R_IDX + _PERM[j], 1)],
                        slab.at[pl.ds(_R_IDX - _owner * _RPW + j, 1)])
        stores[k].start()
    for c in stores:
        c.wait()


def kernel(x, y):
    return (_sc_permute(x), y)
